# baseline (speedup 1.0000x reference)
import functools
import math

import jax
import jax.numpy as jnp
from jax import lax
from jax.experimental import pallas as pl
from jax.experimental.pallas import tpu as pltpu
from jax.experimental.pallas import tpu_sc as plsc

H = 64
HH = 32
NG = 50
CUTOFF = 5.0
NI = 3
NZ = 100
N = 50000
E = 800000

NC = 2
NS = 16
LN = 16

EPW = 25088
E_D = NC * NS * EPW
DW = 512
NWIN = EPW // DW


TAB = 4096
DSC = (TAB - 1) / CUTOFF


def _sc_dist_body(px_ref, py_ref, pz_ref, i_ref, j_ref, out_ref,
                  coord, acc, qcc, ib, jb):
    c = lax.axis_index("c")
    s = lax.axis_index("s")
    base = (s * NC + c) * EPW

    for p, cref in enumerate((px_ref, py_ref, pz_ref)):
        pltpu.sync_copy(cref, coord)

        def win(w, _, p=p):
            e0 = base + w * DW
            pltpu.sync_copy(i_ref.at[pl.ds(e0, DW)], ib)
            pltpu.sync_copy(j_ref.at[pl.ds(e0, DW)], jb)
            for k in range(DW // LN):
                sl = pl.ds(w * DW + k * LN, LN)
                iv = jnp.minimum(ib[pl.ds(k * LN, LN)], N - 1)
                jv = jnp.minimum(jb[pl.ds(k * LN, LN)], N - 1)
                dfr = plsc.load_gather(coord, [jv]) - plsc.load_gather(coord, [iv])
                sq = dfr * dfr
                if p == 0:
                    acc[sl] = sq
                else:
                    acc[sl] = acc[sl] + sq
            return 0

        lax.fori_loop(0, NWIN, win, 0)

    def quant(k, _):
        sl = pl.ds(k * LN, LN)
        x = acc[sl]
        yi = jnp.int32(0x5F3759DF) - lax.shift_right_logical(
            plsc.bitcast(x, jnp.int32), 1)
        y = plsc.bitcast(yi, jnp.float32)
        y = y * (1.5 - 0.5 * x * y * y)
        y = y * (1.5 - 0.5 * x * y * y)
        d = x * y
        q = (d * DSC + 0.5).astype(jnp.int32)
        q = jnp.minimum(jnp.maximum(q, 0), TAB - 1)
        qcc[sl] = q
        return 0

    lax.fori_loop(0, EPW // LN, quant, 0)
    pltpu.sync_copy(qcc, out_ref.at[pl.ds(base, EPW)])


def _sc_dist(px, py, pz, i_d, j_d):
    mesh = plsc.VectorSubcoreMesh(core_axis_name="c", subcore_axis_name="s")
    return pl.kernel(
        _sc_dist_body,
        out_type=jax.ShapeDtypeStruct((E_D,), jnp.int32),
        mesh=mesh,
        scratch_types=[
            pltpu.VMEM((N,), jnp.float32),
            pltpu.VMEM((EPW,), jnp.float32),
            pltpu.VMEM((EPW,), jnp.int32),
            pltpu.VMEM((DW,), jnp.int32),
            pltpu.VMEM((DW,), jnp.int32),
        ],
        compiler_params=pltpu.CompilerParams(needs_layout_passes=False),
    )(px, py, pz, i_d, j_d)


EPT = E_D // NS
GW = 128
SUP = 256
WPS = SUP // GW
NSUP = EPT // SUP
WFR = E_D // 4
NP = 50048
RPT = NP // NS
ZR = 184


def _sc_edge_body(t, ytab_ref, tab_ref, qd_ref, iw_ref, jw_ref, out_ref,
                  acc, tabsp, ibl, jbl, qbl, ib2, jadj, qb2, wfb, rows, sem):
    c = lax.axis_index("c")
    s = lax.axis_index("s")
    cN = c * N

    zero = jnp.zeros((LN,), jnp.float32)
    for r in range(ZR):
        rows[r, pl.ds(0, LN)] = zero
        rows[r, pl.ds(LN, LN)] = zero
    for rb in range(RPT // ZR):
        pltpu.sync_copy(rows.at[pl.ds(0, ZR)],
                        acc.at[pl.ds(s * RPT + rb * ZR, ZR)])
    TPT = TAB // NS
    pltpu.sync_copy(tab_ref.at[t, c, pl.ds(s * TPT, TPT)],
                    rows.at[pl.ds(0, TPT)])
    pltpu.sync_copy(rows.at[pl.ds(0, TPT)], tabsp.at[pl.ds(s * TPT, TPT)])
    plsc.subcore_barrier()

    def sup(sw, _):
        e0 = s * EPT + sw * SUP
        pltpu.sync_copy(iw_ref.at[pl.ds(e0, SUP)], ibl)
        pltpu.sync_copy(jw_ref.at[pl.ds(e0, SUP)], jbl)
        pltpu.sync_copy(qd_ref.at[pl.ds(e0, SUP)], qbl)
        for w in range(WPS):
            for q in range(GW // LN):
                f0 = w * GW + q * LN
                ib2[w, pl.ds(q * LN, LN)] = ibl[pl.ds(f0, LN)]
                jadj[w, pl.ds(q * LN, LN)] = jbl[pl.ds(f0, LN)] + cN
                qb2[w, pl.ds(q * LN, LN)] = qbl[pl.ds(f0, LN)]
        cps = [
            pltpu.async_copy(ytab_ref.at[jadj.at[w]],
                             rows.at[pl.ds(w * GW, GW)], sem)
            for w in range(WPS)
        ] + [
            pltpu.async_copy(tab_ref.at[t, c].at[qb2.at[w]],
                             wfb.at[pl.ds(w * GW, GW)], sem)
            for w in range(WPS)
        ]
        for cp in cps:
            cp.wait()

        def mul(q, _):
            for dr in range(8):
                ri = q * 8 + dr
                for h0 in (0, LN):
                    rows[ri, pl.ds(h0, LN)] = (
                        rows[ri, pl.ds(h0, LN)] * wfb[ri, pl.ds(h0, LN)])
            return 0

        lax.fori_loop(0, SUP // 8, mul, 0)
        for w in range(WPS):
            pltpu.sync_copy(rows.at[pl.ds(w * GW, GW)], acc.at[ib2.at[w]],
                            add=True)
        return 0

    lax.fori_loop(0, NSUP, sup, 0)
    plsc.subcore_barrier()
    pltpu.sync_copy(acc.at[pl.ds(s * RPT, RPT)],
                    out_ref.at[c, pl.ds(s * RPT, RPT)])


def _sc_edge(t, ytab, tab, qd, iw, jw):
    mesh = plsc.VectorSubcoreMesh(core_axis_name="c", subcore_axis_name="s")
    return pl.kernel(
        functools.partial(_sc_edge_body, t),
        out_type=jax.ShapeDtypeStruct((NC, NP, HH), jnp.float32),
        mesh=mesh,
        scratch_types=[
            pltpu.VMEM_SHARED((NP, HH), jnp.float32),
            pltpu.VMEM_SHARED((TAB, HH), jnp.float32),
            pltpu.VMEM((SUP,), jnp.int32),
            pltpu.VMEM((SUP,), jnp.int32),
            pltpu.VMEM((SUP,), jnp.int32),
            pltpu.VMEM((WPS, GW), jnp.int32),
            pltpu.VMEM((WPS, GW), jnp.int32),
            pltpu.VMEM((WPS, GW), jnp.int32),
            pltpu.VMEM((SUP, HH), jnp.float32),
            pltpu.VMEM((SUP, HH), jnp.float32),
            pltpu.SemaphoreType.DMA,
        ],
        compiler_params=pltpu.CompilerParams(needs_layout_passes=False,
                                             use_tc_tiling_on_sc=False),
    )(ytab, tab, qd, iw, jw)


def _ssp(x):
    return jax.nn.softplus(x) - math.log(2.0)


def _tc_table_body(fw1_ref, fb1_ref, fw2_ref, fb2_ref, out_ref):
    d = lax.broadcasted_iota(jnp.int32, (TAB,), 0).astype(jnp.float32) / DSC
    width = CUTOFF / (NG - 1)
    offs = lax.broadcasted_iota(jnp.int32, (1, NG), 1).astype(jnp.float32) * width
    delta = d[:, None] - offs
    rbf = jnp.exp((-0.5 / (width * width)) * delta * delta)
    fcut = 0.5 * (jnp.cos(d * (math.pi / CUTOFF)) + 1.0)
    fcut = fcut * (d < CUTOFF).astype(jnp.float32)
    for t in range(NI):
        h1 = _ssp(jnp.dot(rbf, fw1_ref[t], preferred_element_type=jnp.float32)
                  + fb1_ref[t])
        wfv = jnp.dot(h1, fw2_ref[t], preferred_element_type=jnp.float32) + fb2_ref[t]
        wfv = wfv * fcut[:, None]
        out_ref[t, 0] = wfv[:, :HH]
        out_ref[t, 1] = wfv[:, HH:]


def _tc_table(fw1, fb1, fw2, fb2):
    return pl.pallas_call(
        _tc_table_body,
        out_shape=jax.ShapeDtypeStruct((NI, 2, TAB, HH), jnp.float32),
    )(fw1, fb1, fw2, fb2)


BN = 2000


def _tc_node0_body(z_ref, emb_ref, inw_ref, x_ref, y2_ref):
    z = z_ref[0, 0, :]
    zi = lax.broadcasted_iota(jnp.int32, (BN, NZ), 1)
    oh = (z[:, None] == zi).astype(jnp.float32)
    x = jnp.dot(oh, emb_ref[...], preferred_element_type=jnp.float32)
    y = jnp.dot(x, inw_ref[...], preferred_element_type=jnp.float32)
    x_ref[...] = x
    y2_ref[0] = y[:, :HH]
    y2_ref[1] = y[:, HH:]


def _tc_node0(z, emb, inw):
    return pl.pallas_call(
        _tc_node0_body,
        grid=(N // BN,),
        in_specs=[
            pl.BlockSpec((1, 1, BN), lambda n: (n, 0, 0)),
            pl.BlockSpec((NZ, H), lambda n: (0, 0)),
            pl.BlockSpec((H, H), lambda n: (0, 0)),
        ],
        out_specs=[
            pl.BlockSpec((BN, H), lambda n: (n, 0)),
            pl.BlockSpec((2, BN, HH), lambda n: (0, n, 0)),
        ],
        out_shape=[
            jax.ShapeDtypeStruct((N, H), jnp.float32),
            jax.ShapeDtypeStruct((2, N, HH), jnp.float32),
        ],
    )(z, emb, inw)


def _node_update(agg2_ref, x_ref, f2w_ref, f2b_ref, ow_ref, ob_ref):
    pre = (jnp.dot(agg2_ref[0], f2w_ref[:HH, :], preferred_element_type=jnp.float32)
           + jnp.dot(agg2_ref[1], f2w_ref[HH:, :], preferred_element_type=jnp.float32)
           + f2b_ref[...])
    v = jnp.dot(_ssp(pre), ow_ref[...], preferred_element_type=jnp.float32) + ob_ref[...]
    return x_ref[...] + v


def _tc_node_body(agg2_ref, x_ref, f2w_ref, f2b_ref, ow_ref, ob_ref, inw_ref,
                  xn_ref, y2_ref):
    xn = _node_update(agg2_ref, x_ref, f2w_ref, f2b_ref, ow_ref, ob_ref)
    xn_ref[...] = xn
    y = jnp.dot(xn, inw_ref[...], preferred_element_type=jnp.float32)
    y2_ref[0] = y[:, :HH]
    y2_ref[1] = y[:, HH:]


def _tc_node(agg2, x, f2w, f2b, ow, ob, inw):
    return pl.pallas_call(
        _tc_node_body,
        grid=(N // BN,),
        in_specs=[
            pl.BlockSpec((2, BN, HH), lambda n: (0, n, 0)),
            pl.BlockSpec((BN, H), lambda n: (n, 0)),
            pl.BlockSpec((H, H), lambda n: (0, 0)),
            pl.BlockSpec((H,), lambda n: (0,)),
            pl.BlockSpec((H, H), lambda n: (0, 0)),
            pl.BlockSpec((H,), lambda n: (0,)),
            pl.BlockSpec((H, H), lambda n: (0, 0)),
        ],
        out_specs=[
            pl.BlockSpec((BN, H), lambda n: (n, 0)),
            pl.BlockSpec((2, BN, HH), lambda n: (0, n, 0)),
        ],
        out_shape=[
            jax.ShapeDtypeStruct((N, H), jnp.float32),
            jax.ShapeDtypeStruct((2, N, HH), jnp.float32),
        ],
    )(agg2, x, f2w, f2b, ow, ob, inw)


def _tc_node2_body(agg2_ref, x_ref, f2w_ref, f2b_ref, ow_ref, ob_ref,
                   w1_ref, b1_ref, w2_ref, b2_ref, out_ref):
    xn = _node_update(agg2_ref, x_ref, f2w_ref, f2b_ref, ow_ref, ob_ref)
    h = _ssp(jnp.dot(xn, w1_ref[...], preferred_element_type=jnp.float32) + b1_ref[...])
    out_ref[...] = jnp.dot(h, w2_ref[...], preferred_element_type=jnp.float32) + b2_ref[...]


def _tc_node2(agg2, x, f2w, f2b, ow, ob, w1, b1, w2, b2):
    return pl.pallas_call(
        _tc_node2_body,
        grid=(N // BN,),
        in_specs=[
            pl.BlockSpec((2, BN, HH), lambda n: (0, n, 0)),
            pl.BlockSpec((BN, H), lambda n: (n, 0)),
            pl.BlockSpec((H, H), lambda n: (0, 0)),
            pl.BlockSpec((H,), lambda n: (0,)),
            pl.BlockSpec((H, H), lambda n: (0, 0)),
            pl.BlockSpec((H,), lambda n: (0,)),
            pl.BlockSpec((H, H // 2), lambda n: (0, 0)),
            pl.BlockSpec((H // 2,), lambda n: (0,)),
            pl.BlockSpec((H // 2, 3), lambda n: (0, 0)),
            pl.BlockSpec((3,), lambda n: (0,)),
        ],
        out_specs=pl.BlockSpec((BN, 3), lambda n: (n, 0)),
        out_shape=jax.ShapeDtypeStruct((N, 3), jnp.float32),
    )(agg2, x, f2w, f2b, ow, ob, w1, b1, w2, b2)


def kernel(z, pos, edge_index, batch, params):
    i = edge_index[0].astype(jnp.int32)
    j = edge_index[1].astype(jnp.int32)
    pad = E_D - E
    i_d = jnp.concatenate([i, N + (jnp.arange(pad, dtype=jnp.int32) % (NP - N))])
    j_d = jnp.concatenate([j, jnp.zeros((pad,), jnp.int32)])

    qd = _sc_dist(pos[:, 0], pos[:, 1], pos[:, 2], i_d, j_d)

    inter = params['interactions']
    fw1 = jnp.stack([p['fw1'] for p in inter])
    fb1 = jnp.stack([p['fb1'] for p in inter])
    fw2 = jnp.stack([p['fw2'] for p in inter])
    fb2 = jnp.stack([p['fb2'] for p in inter])
    tab = _tc_table(fw1, fb1, fw2, fb2)

    x, y2 = _tc_node0(z.astype(jnp.int32).reshape(N // BN, 1, BN),
                      params['emb'], inter[0]['inw'])
    score = None
    for t in range(NI):
        ytab = y2.reshape(NC * N, HH)
        agg2 = _sc_edge(t, ytab, tab, qd, i_d, j_d)
        p = inter[t]
        if t < NI - 1:
            x, y2 = _tc_node(agg2, x, p['f2w'], p['f2b'], p['ow'], p['ob'],
                             inter[t + 1]['inw'])
        else:
            score = _tc_node2(agg2, x, p['f2w'], p['f2b'], p['ow'], p['ob'],
                              params['out_w1'], params['out_b1'],
                              params['out_w2'], params['out_b2'])
    return score


# speedup vs baseline: 7.5805x; 7.5805x over previous
import functools
import math

import jax
import jax.numpy as jnp
from jax import lax
from jax.experimental import pallas as pl
from jax.experimental.pallas import tpu as pltpu
from jax.experimental.pallas import tpu_sc as plsc

H = 64
HH = 32
NG = 50
CUTOFF = 5.0
NI = 3
NZ = 100
N = 50000
E = 800000

NC = 2
NS = 16
LN = 16

EPW = 25088
E_D = NC * NS * EPW
DW = 512
NWIN = EPW // DW


TAB = 4096
DSC = (TAB - 1) / CUTOFF


def _sc_dist_body(px_ref, py_ref, pz_ref, i_ref, j_ref, out_ref,
                  coord, acc, qcc, ib, jb):
    c = lax.axis_index("c")
    s = lax.axis_index("s")
    base = (s * NC + c) * EPW

    for p, cref in enumerate((px_ref, py_ref, pz_ref)):
        pltpu.sync_copy(cref, coord)

        def win(w, _, p=p):
            e0 = base + w * DW
            pltpu.sync_copy(i_ref.at[pl.ds(e0, DW)], ib)
            pltpu.sync_copy(j_ref.at[pl.ds(e0, DW)], jb)
            for k in range(DW // LN):
                sl = pl.ds(w * DW + k * LN, LN)
                iv = jnp.minimum(ib[pl.ds(k * LN, LN)], N - 1)
                jv = jnp.minimum(jb[pl.ds(k * LN, LN)], N - 1)
                dfr = plsc.load_gather(coord, [jv]) - plsc.load_gather(coord, [iv])
                sq = dfr * dfr
                if p == 0:
                    acc[sl] = sq
                else:
                    acc[sl] = acc[sl] + sq
            return 0

        lax.fori_loop(0, NWIN, win, 0)

    def quant(k, _):
        sl = pl.ds(k * LN, LN)
        x = acc[sl]
        yi = jnp.int32(0x5F3759DF) - lax.shift_right_logical(
            plsc.bitcast(x, jnp.int32), 1)
        y = plsc.bitcast(yi, jnp.float32)
        y = y * (1.5 - 0.5 * x * y * y)
        y = y * (1.5 - 0.5 * x * y * y)
        d = x * y
        q = (d * DSC + 0.5).astype(jnp.int32)
        q = jnp.minimum(jnp.maximum(q, 0), TAB - 1)
        qcc[sl] = q
        return 0

    lax.fori_loop(0, EPW // LN, quant, 0)
    pltpu.sync_copy(qcc, out_ref.at[pl.ds(base, EPW)])


def _sc_dist(px, py, pz, i_d, j_d):
    mesh = plsc.VectorSubcoreMesh(core_axis_name="c", subcore_axis_name="s")
    return pl.kernel(
        _sc_dist_body,
        out_type=jax.ShapeDtypeStruct((E_D,), jnp.int32),
        mesh=mesh,
        scratch_types=[
            pltpu.VMEM((N,), jnp.float32),
            pltpu.VMEM((EPW,), jnp.float32),
            pltpu.VMEM((EPW,), jnp.int32),
            pltpu.VMEM((DW,), jnp.int32),
            pltpu.VMEM((DW,), jnp.int32),
        ],
        compiler_params=pltpu.CompilerParams(needs_layout_passes=False),
    )(px, py, pz, i_d, j_d)


EPT = E_D // NS
GW = 128
SUP = 256
WPS = SUP // GW
NSUP = EPT // SUP
WFR = E_D // 4
NP = 50048
RPT = NP // NS
ZR = 184


def _sc_edge_body(t, ytab_ref, tab_ref, qd_ref, iw_ref, jw_ref, out_ref,
                  acc, tabsp, ibl, jbl, qbl, ib2, jadj, qb2, wfb, rows, sem, sem2):
    c = lax.axis_index("c")
    s = lax.axis_index("s")
    cN = c * N

    zero = jnp.zeros((LN,), jnp.float32)
    for r in range(ZR):
        rows[r, pl.ds(0, LN)] = zero
        rows[r, pl.ds(LN, LN)] = zero
    for rb in range(RPT // ZR):
        pltpu.sync_copy(rows.at[pl.ds(0, ZR)],
                        acc.at[pl.ds(s * RPT + rb * ZR, ZR)])
    TPT = TAB // NS
    pltpu.sync_copy(tab_ref.at[t, c, pl.ds(s * TPT, TPT)],
                    rows.at[pl.ds(0, TPT)])
    pltpu.sync_copy(rows.at[pl.ds(0, TPT)], tabsp.at[pl.ds(s * TPT, TPT)])
    plsc.subcore_barrier()

    def sup(sw, _):
        e0 = s * EPT + sw * SUP
        pltpu.sync_copy(iw_ref.at[pl.ds(e0, SUP)], ibl)
        pltpu.sync_copy(jw_ref.at[pl.ds(e0, SUP)], jbl)
        pltpu.sync_copy(qd_ref.at[pl.ds(e0, SUP)], qbl)
        for w in range(WPS):
            for q in range(GW // LN):
                f0 = w * GW + q * LN
                ib2[w, pl.ds(q * LN, LN)] = ibl[pl.ds(f0, LN)]
                jadj[w, pl.ds(q * LN, LN)] = jbl[pl.ds(f0, LN)] + cN
                qb2[w, pl.ds(q * LN, LN)] = qbl[pl.ds(f0, LN)]
        cps = [
            pltpu.async_copy(ytab_ref.at[jadj.at[w]],
                             rows.at[pl.ds(w * GW, GW)], sem)
            for w in range(WPS)
        ] + [
            pltpu.async_copy(tabsp.at[qb2.at[w]],
                             wfb.at[pl.ds(w * GW, GW)], sem2)
            for w in range(WPS)
        ]
        for cp in cps:
            cp.wait()

        def mul(q, _):
            for dr in range(8):
                ri = q * 8 + dr
                for h0 in (0, LN):
                    rows[ri, pl.ds(h0, LN)] = (
                        rows[ri, pl.ds(h0, LN)] * wfb[ri, pl.ds(h0, LN)])
            return 0

        lax.fori_loop(0, SUP // 8, mul, 0)
        for w in range(WPS):
            pltpu.sync_copy(rows.at[pl.ds(w * GW, GW)], acc.at[ib2.at[w]],
                            add=True)
        return 0

    lax.fori_loop(0, NSUP, sup, 0)
    plsc.subcore_barrier()
    pltpu.sync_copy(acc.at[pl.ds(s * RPT, RPT)],
                    out_ref.at[c, pl.ds(s * RPT, RPT)])


def _sc_edge(t, ytab, tab, qd, iw, jw):
    mesh = plsc.VectorSubcoreMesh(core_axis_name="c", subcore_axis_name="s")
    return pl.kernel(
        functools.partial(_sc_edge_body, t),
        out_type=jax.ShapeDtypeStruct((NC, NP, HH), jnp.float32),
        mesh=mesh,
        scratch_types=[
            pltpu.VMEM_SHARED((NP, HH), jnp.float32),
            pltpu.VMEM_SHARED((TAB, HH), jnp.float32),
            pltpu.VMEM((SUP,), jnp.int32),
            pltpu.VMEM((SUP,), jnp.int32),
            pltpu.VMEM((SUP,), jnp.int32),
            pltpu.VMEM((WPS, GW), jnp.int32),
            pltpu.VMEM((WPS, GW), jnp.int32),
            pltpu.VMEM((WPS, GW), jnp.int32),
            pltpu.VMEM((SUP, HH), jnp.float32),
            pltpu.VMEM((SUP, HH), jnp.float32),
            pltpu.SemaphoreType.DMA,
            pltpu.SemaphoreType.DMA,
        ],
        compiler_params=pltpu.CompilerParams(needs_layout_passes=False,
                                             use_tc_tiling_on_sc=False),
    )(ytab, tab, qd, iw, jw)


def _ssp(x):
    return jax.nn.softplus(x) - math.log(2.0)


def _tc_table_body(fw1_ref, fb1_ref, fw2_ref, fb2_ref, out_ref):
    d = lax.broadcasted_iota(jnp.int32, (TAB,), 0).astype(jnp.float32) / DSC
    width = CUTOFF / (NG - 1)
    offs = lax.broadcasted_iota(jnp.int32, (1, NG), 1).astype(jnp.float32) * width
    delta = d[:, None] - offs
    rbf = jnp.exp((-0.5 / (width * width)) * delta * delta)
    fcut = 0.5 * (jnp.cos(d * (math.pi / CUTOFF)) + 1.0)
    fcut = fcut * (d < CUTOFF).astype(jnp.float32)
    for t in range(NI):
        h1 = _ssp(jnp.dot(rbf, fw1_ref[t], preferred_element_type=jnp.float32)
                  + fb1_ref[t])
        wfv = jnp.dot(h1, fw2_ref[t], preferred_element_type=jnp.float32) + fb2_ref[t]
        wfv = wfv * fcut[:, None]
        out_ref[t, 0] = wfv[:, :HH]
        out_ref[t, 1] = wfv[:, HH:]


def _tc_table(fw1, fb1, fw2, fb2):
    return pl.pallas_call(
        _tc_table_body,
        out_shape=jax.ShapeDtypeStruct((NI, 2, TAB, HH), jnp.float32),
    )(fw1, fb1, fw2, fb2)


BN = 2000


def _tc_node0_body(z_ref, emb_ref, inw_ref, x_ref, y2_ref):
    z = z_ref[0, 0, :]
    zi = lax.broadcasted_iota(jnp.int32, (BN, NZ), 1)
    oh = (z[:, None] == zi).astype(jnp.float32)
    x = jnp.dot(oh, emb_ref[...], preferred_element_type=jnp.float32)
    y = jnp.dot(x, inw_ref[...], preferred_element_type=jnp.float32)
    x_ref[...] = x
    y2_ref[0] = y[:, :HH]
    y2_ref[1] = y[:, HH:]


def _tc_node0(z, emb, inw):
    return pl.pallas_call(
        _tc_node0_body,
        grid=(N // BN,),
        in_specs=[
            pl.BlockSpec((1, 1, BN), lambda n: (n, 0, 0)),
            pl.BlockSpec((NZ, H), lambda n: (0, 0)),
            pl.BlockSpec((H, H), lambda n: (0, 0)),
        ],
        out_specs=[
            pl.BlockSpec((BN, H), lambda n: (n, 0)),
            pl.BlockSpec((2, BN, HH), lambda n: (0, n, 0)),
        ],
        out_shape=[
            jax.ShapeDtypeStruct((N, H), jnp.float32),
            jax.ShapeDtypeStruct((2, N, HH), jnp.float32),
        ],
    )(z, emb, inw)


def _node_update(agg2_ref, x_ref, f2w_ref, f2b_ref, ow_ref, ob_ref):
    pre = (jnp.dot(agg2_ref[0], f2w_ref[:HH, :], preferred_element_type=jnp.float32)
           + jnp.dot(agg2_ref[1], f2w_ref[HH:, :], preferred_element_type=jnp.float32)
           + f2b_ref[...])
    v = jnp.dot(_ssp(pre), ow_ref[...], preferred_element_type=jnp.float32) + ob_ref[...]
    return x_ref[...] + v


def _tc_node_body(agg2_ref, x_ref, f2w_ref, f2b_ref, ow_ref, ob_ref, inw_ref,
                  xn_ref, y2_ref):
    xn = _node_update(agg2_ref, x_ref, f2w_ref, f2b_ref, ow_ref, ob_ref)
    xn_ref[...] = xn
    y = jnp.dot(xn, inw_ref[...], preferred_element_type=jnp.float32)
    y2_ref[0] = y[:, :HH]
    y2_ref[1] = y[:, HH:]


def _tc_node(agg2, x, f2w, f2b, ow, ob, inw):
    return pl.pallas_call(
        _tc_node_body,
        grid=(N // BN,),
        in_specs=[
            pl.BlockSpec((2, BN, HH), lambda n: (0, n, 0)),
            pl.BlockSpec((BN, H), lambda n: (n, 0)),
            pl.BlockSpec((H, H), lambda n: (0, 0)),
            pl.BlockSpec((H,), lambda n: (0,)),
            pl.BlockSpec((H, H), lambda n: (0, 0)),
            pl.BlockSpec((H,), lambda n: (0,)),
            pl.BlockSpec((H, H), lambda n: (0, 0)),
        ],
        out_specs=[
            pl.BlockSpec((BN, H), lambda n: (n, 0)),
            pl.BlockSpec((2, BN, HH), lambda n: (0, n, 0)),
        ],
        out_shape=[
            jax.ShapeDtypeStruct((N, H), jnp.float32),
            jax.ShapeDtypeStruct((2, N, HH), jnp.float32),
        ],
    )(agg2, x, f2w, f2b, ow, ob, inw)


def _tc_node2_body(agg2_ref, x_ref, f2w_ref, f2b_ref, ow_ref, ob_ref,
                   w1_ref, b1_ref, w2_ref, b2_ref, out_ref):
    xn = _node_update(agg2_ref, x_ref, f2w_ref, f2b_ref, ow_ref, ob_ref)
    h = _ssp(jnp.dot(xn, w1_ref[...], preferred_element_type=jnp.float32) + b1_ref[...])
    out_ref[...] = jnp.dot(h, w2_ref[...], preferred_element_type=jnp.float32) + b2_ref[...]


def _tc_node2(agg2, x, f2w, f2b, ow, ob, w1, b1, w2, b2):
    return pl.pallas_call(
        _tc_node2_body,
        grid=(N // BN,),
        in_specs=[
            pl.BlockSpec((2, BN, HH), lambda n: (0, n, 0)),
            pl.BlockSpec((BN, H), lambda n: (n, 0)),
            pl.BlockSpec((H, H), lambda n: (0, 0)),
            pl.BlockSpec((H,), lambda n: (0,)),
            pl.BlockSpec((H, H), lambda n: (0, 0)),
            pl.BlockSpec((H,), lambda n: (0,)),
            pl.BlockSpec((H, H // 2), lambda n: (0, 0)),
            pl.BlockSpec((H // 2,), lambda n: (0,)),
            pl.BlockSpec((H // 2, 3), lambda n: (0, 0)),
            pl.BlockSpec((3,), lambda n: (0,)),
        ],
        out_specs=pl.BlockSpec((BN, 3), lambda n: (n, 0)),
        out_shape=jax.ShapeDtypeStruct((N, 3), jnp.float32),
    )(agg2, x, f2w, f2b, ow, ob, w1, b1, w2, b2)


def kernel(z, pos, edge_index, batch, params):
    i = edge_index[0].astype(jnp.int32)
    j = edge_index[1].astype(jnp.int32)
    pad = E_D - E
    i_d = jnp.concatenate([i, N + (jnp.arange(pad, dtype=jnp.int32) % (NP - N))])
    j_d = jnp.concatenate([j, jnp.zeros((pad,), jnp.int32)])

    qd = _sc_dist(pos[:, 0], pos[:, 1], pos[:, 2], i_d, j_d)

    inter = params['interactions']
    fw1 = jnp.stack([p['fw1'] for p in inter])
    fb1 = jnp.stack([p['fb1'] for p in inter])
    fw2 = jnp.stack([p['fw2'] for p in inter])
    fb2 = jnp.stack([p['fb2'] for p in inter])
    tab = _tc_table(fw1, fb1, fw2, fb2)

    x, y2 = _tc_node0(z.astype(jnp.int32).reshape(N // BN, 1, BN),
                      params['emb'], inter[0]['inw'])
    score = None
    for t in range(NI):
        ytab = y2.reshape(NC * N, HH)
        agg2 = _sc_edge(t, ytab, tab, qd, i_d, j_d)
        p = inter[t]
        if t < NI - 1:
            x, y2 = _tc_node(agg2, x, p['f2w'], p['f2b'], p['ow'], p['ob'],
                             inter[t + 1]['inw'])
        else:
            score = _tc_node2(agg2, x, p['f2w'], p['f2b'], p['ow'], p['ob'],
                              params['out_w1'], params['out_b1'],
                              params['out_w2'], params['out_b2'])
    return score


# speedup vs baseline: 9.3079x; 1.2279x over previous
import functools
import math

import jax
import jax.numpy as jnp
from jax import lax
from jax.experimental import pallas as pl
from jax.experimental.pallas import tpu as pltpu
from jax.experimental.pallas import tpu_sc as plsc

H = 64
HH = 32
NG = 50
CUTOFF = 5.0
NI = 3
NZ = 100
N = 50000
E = 800000

NC = 2
NS = 16
LN = 16

EPW = 25088
E_D = NC * NS * EPW
DW = 512
NWIN = EPW // DW


TAB = 4096
DSC = (TAB - 1) / CUTOFF


def _sc_dist_body(px_ref, py_ref, pz_ref, i_ref, j_ref, out_ref,
                  coord, acc, qcc, ib, jb):
    c = lax.axis_index("c")
    s = lax.axis_index("s")
    base = (s * NC + c) * EPW

    for p, cref in enumerate((px_ref, py_ref, pz_ref)):
        pltpu.sync_copy(cref, coord)

        def win(w, _, p=p):
            e0 = base + w * DW
            pltpu.sync_copy(i_ref.at[pl.ds(e0, DW)], ib)
            pltpu.sync_copy(j_ref.at[pl.ds(e0, DW)], jb)
            for k in range(DW // LN):
                sl = pl.ds(w * DW + k * LN, LN)
                iv = jnp.minimum(ib[pl.ds(k * LN, LN)], N - 1)
                jv = jnp.minimum(jb[pl.ds(k * LN, LN)], N - 1)
                dfr = plsc.load_gather(coord, [jv]) - plsc.load_gather(coord, [iv])
                sq = dfr * dfr
                if p == 0:
                    acc[sl] = sq
                else:
                    acc[sl] = acc[sl] + sq
            return 0

        lax.fori_loop(0, NWIN, win, 0)

    def quant(k, _):
        sl = pl.ds(k * LN, LN)
        x = acc[sl]
        yi = jnp.int32(0x5F3759DF) - lax.shift_right_logical(
            plsc.bitcast(x, jnp.int32), 1)
        y = plsc.bitcast(yi, jnp.float32)
        y = y * (1.5 - 0.5 * x * y * y)
        y = y * (1.5 - 0.5 * x * y * y)
        d = x * y
        q = (d * DSC + 0.5).astype(jnp.int32)
        q = jnp.minimum(jnp.maximum(q, 0), TAB - 1)
        qcc[sl] = q
        return 0

    lax.fori_loop(0, EPW // LN, quant, 0)
    pltpu.sync_copy(qcc, out_ref.at[pl.ds(base, EPW)])


def _sc_dist(px, py, pz, i_d, j_d):
    mesh = plsc.VectorSubcoreMesh(core_axis_name="c", subcore_axis_name="s")
    return pl.kernel(
        _sc_dist_body,
        out_type=jax.ShapeDtypeStruct((E_D,), jnp.int32),
        mesh=mesh,
        scratch_types=[
            pltpu.VMEM((N,), jnp.float32),
            pltpu.VMEM((EPW,), jnp.float32),
            pltpu.VMEM((EPW,), jnp.int32),
            pltpu.VMEM((DW,), jnp.int32),
            pltpu.VMEM((DW,), jnp.int32),
        ],
        compiler_params=pltpu.CompilerParams(needs_layout_passes=False),
    )(px, py, pz, i_d, j_d)


EPT = E_D // NS
GW = 128
NSUP = EPT // GW
NGRP = NSUP // 8
NP = 50048
RPT = NP // NS
ZR = 184


def _sc_edge_body(t, ytab_ref, tab_ref, qd_ref, ij_ref, out_ref,
                  acc, tabsp, ijg, qg, ib2, jadj, wfb, rows,
                  semh0, semh1, sems0, sems1):
    c = lax.axis_index("c")
    s = lax.axis_index("s")
    cN = c * N
    semh = (semh0, semh1)
    sems = (sems0, sems1)

    zero = jnp.zeros((LN,), jnp.float32)
    for r in range(ZR):
        rows[r, pl.ds(0, LN)] = zero
        rows[r, pl.ds(LN, LN)] = zero
    for rb in range(RPT // ZR):
        pltpu.sync_copy(rows.at[pl.ds(0, ZR)],
                        acc.at[pl.ds(s * RPT + rb * ZR, ZR)])
    TPT = TAB // NS
    pltpu.sync_copy(tab_ref.at[t, c, pl.ds(s * TPT, TPT)],
                    rows.at[pl.ds(0, TPT)])
    pltpu.sync_copy(rows.at[pl.ds(0, TPT)], tabsp.at[pl.ds(s * TPT, TPT)])
    plsc.subcore_barrier()

    row_base = s * NSUP

    def unpack_fire(gsel, wr, dst):
        for v in range(GW // LN):
            ij = ijg[gsel, wr, pl.ds(v * LN, LN)]
            ib2[dst, pl.ds(v * LN, LN)] = ij & 0xFFFF
            jadj[dst, pl.ds(v * LN, LN)] = lax.shift_right_logical(ij, 16) + cN
        cph = pltpu.async_copy(ytab_ref.at[jadj.at[dst]],
                               rows.at[pl.ds(dst * GW, GW)], semh[dst])
        cps = pltpu.async_copy(tabsp.at[qg.at[gsel, wr]],
                               wfb.at[pl.ds(dst * GW, GW)], sems[dst])
        return cph, cps

    def load_group(g, gsel):
        pltpu.sync_copy(ij_ref.at[pl.ds(row_base + g * 8, 8)], ijg.at[gsel])
        pltpu.sync_copy(qd_ref.at[pl.ds(row_base + g * 8, 8)], qg.at[gsel])

    load_group(0, 0)
    unpack_fire(0, 0, 0)

    def grp(g, _):
        gsel = g & 1
        pl.when(g < NGRP - 1)(lambda: load_group(g + 1, 1 - (g & 1)))
        for w in range(8):
            s0 = w & 1
            s1 = 1 - s0
            if w < 7:
                unpack_fire(gsel, w + 1, s1)
            else:
                pl.when(g < NGRP - 1)(
                    lambda: unpack_fire(1 - (g & 1), 0, s1) and None)
            pltpu.make_async_copy(ytab_ref.at[jadj.at[s0]],
                                  rows.at[pl.ds(s0 * GW, GW)], semh[s0]).wait()
            pltpu.make_async_copy(tabsp.at[qg.at[gsel, w]],
                                  wfb.at[pl.ds(s0 * GW, GW)], sems[s0]).wait()

            def mul(q, _):
                for dr in range(8):
                    ri = s0 * GW + q * 8 + dr
                    for h0 in (0, LN):
                        rows[ri, pl.ds(h0, LN)] = (
                            rows[ri, pl.ds(h0, LN)] * wfb[ri, pl.ds(h0, LN)])
                return 0

            lax.fori_loop(0, GW // 8, mul, 0)
            pltpu.sync_copy(rows.at[pl.ds(s0 * GW, GW)], acc.at[ib2.at[s0]],
                            add=True)
        return 0

    lax.fori_loop(0, NGRP, grp, 0)
    plsc.subcore_barrier()
    pltpu.sync_copy(acc.at[pl.ds(s * RPT, RPT)],
                    out_ref.at[c, pl.ds(s * RPT, RPT)])


def _sc_edge(t, ytab, tab, qd2, ij2):
    mesh = plsc.VectorSubcoreMesh(core_axis_name="c", subcore_axis_name="s")
    return pl.kernel(
        functools.partial(_sc_edge_body, t),
        out_type=jax.ShapeDtypeStruct((NC, NP, HH), jnp.float32),
        mesh=mesh,
        scratch_types=[
            pltpu.VMEM_SHARED((NP, HH), jnp.float32),
            pltpu.VMEM_SHARED((TAB, HH), jnp.float32),
            pltpu.VMEM((2, 8, GW), jnp.int32),
            pltpu.VMEM((2, 8, GW), jnp.int32),
            pltpu.VMEM((2, GW), jnp.int32),
            pltpu.VMEM((2, GW), jnp.int32),
            pltpu.VMEM((2 * GW, HH), jnp.float32),
            pltpu.VMEM((2 * GW, HH), jnp.float32),
            pltpu.SemaphoreType.DMA,
            pltpu.SemaphoreType.DMA,
            pltpu.SemaphoreType.DMA,
            pltpu.SemaphoreType.DMA,
        ],
        compiler_params=pltpu.CompilerParams(needs_layout_passes=False,
                                             use_tc_tiling_on_sc=False),
    )(ytab, tab, qd2, ij2)


def _ssp(x):
    return jax.nn.softplus(x) - math.log(2.0)


def _tc_table_body(fw1_ref, fb1_ref, fw2_ref, fb2_ref, out_ref):
    d = lax.broadcasted_iota(jnp.int32, (TAB,), 0).astype(jnp.float32) / DSC
    width = CUTOFF / (NG - 1)
    offs = lax.broadcasted_iota(jnp.int32, (1, NG), 1).astype(jnp.float32) * width
    delta = d[:, None] - offs
    rbf = jnp.exp((-0.5 / (width * width)) * delta * delta)
    fcut = 0.5 * (jnp.cos(d * (math.pi / CUTOFF)) + 1.0)
    fcut = fcut * (d < CUTOFF).astype(jnp.float32)
    for t in range(NI):
        h1 = _ssp(jnp.dot(rbf, fw1_ref[t], preferred_element_type=jnp.float32)
                  + fb1_ref[t])
        wfv = jnp.dot(h1, fw2_ref[t], preferred_element_type=jnp.float32) + fb2_ref[t]
        wfv = wfv * fcut[:, None]
        out_ref[t, 0] = wfv[:, :HH]
        out_ref[t, 1] = wfv[:, HH:]


def _tc_table(fw1, fb1, fw2, fb2):
    return pl.pallas_call(
        _tc_table_body,
        out_shape=jax.ShapeDtypeStruct((NI, 2, TAB, HH), jnp.float32),
    )(fw1, fb1, fw2, fb2)


BN = 2000


def _tc_node0_body(z_ref, emb_ref, inw_ref, x_ref, y2_ref):
    z = z_ref[0, 0, :]
    zi = lax.broadcasted_iota(jnp.int32, (BN, NZ), 1)
    oh = (z[:, None] == zi).astype(jnp.float32)
    x = jnp.dot(oh, emb_ref[...], preferred_element_type=jnp.float32)
    y = jnp.dot(x, inw_ref[...], preferred_element_type=jnp.float32)
    x_ref[...] = x
    y2_ref[0] = y[:, :HH]
    y2_ref[1] = y[:, HH:]


def _tc_node0(z, emb, inw):
    return pl.pallas_call(
        _tc_node0_body,
        grid=(N // BN,),
        in_specs=[
            pl.BlockSpec((1, 1, BN), lambda n: (n, 0, 0)),
            pl.BlockSpec((NZ, H), lambda n: (0, 0)),
            pl.BlockSpec((H, H), lambda n: (0, 0)),
        ],
        out_specs=[
            pl.BlockSpec((BN, H), lambda n: (n, 0)),
            pl.BlockSpec((2, BN, HH), lambda n: (0, n, 0)),
        ],
        out_shape=[
            jax.ShapeDtypeStruct((N, H), jnp.float32),
            jax.ShapeDtypeStruct((2, N, HH), jnp.float32),
        ],
    )(z, emb, inw)


def _node_update(agg2_ref, x_ref, f2w_ref, f2b_ref, ow_ref, ob_ref):
    pre = (jnp.dot(agg2_ref[0], f2w_ref[:HH, :], preferred_element_type=jnp.float32)
           + jnp.dot(agg2_ref[1], f2w_ref[HH:, :], preferred_element_type=jnp.float32)
           + f2b_ref[...])
    v = jnp.dot(_ssp(pre), ow_ref[...], preferred_element_type=jnp.float32) + ob_ref[...]
    return x_ref[...] + v


def _tc_node_body(agg2_ref, x_ref, f2w_ref, f2b_ref, ow_ref, ob_ref, inw_ref,
                  xn_ref, y2_ref):
    xn = _node_update(agg2_ref, x_ref, f2w_ref, f2b_ref, ow_ref, ob_ref)
    xn_ref[...] = xn
    y = jnp.dot(xn, inw_ref[...], preferred_element_type=jnp.float32)
    y2_ref[0] = y[:, :HH]
    y2_ref[1] = y[:, HH:]


def _tc_node(agg2, x, f2w, f2b, ow, ob, inw):
    return pl.pallas_call(
        _tc_node_body,
        grid=(N // BN,),
        in_specs=[
            pl.BlockSpec((2, BN, HH), lambda n: (0, n, 0)),
            pl.BlockSpec((BN, H), lambda n: (n, 0)),
            pl.BlockSpec((H, H), lambda n: (0, 0)),
            pl.BlockSpec((H,), lambda n: (0,)),
            pl.BlockSpec((H, H), lambda n: (0, 0)),
            pl.BlockSpec((H,), lambda n: (0,)),
            pl.BlockSpec((H, H), lambda n: (0, 0)),
        ],
        out_specs=[
            pl.BlockSpec((BN, H), lambda n: (n, 0)),
            pl.BlockSpec((2, BN, HH), lambda n: (0, n, 0)),
        ],
        out_shape=[
            jax.ShapeDtypeStruct((N, H), jnp.float32),
            jax.ShapeDtypeStruct((2, N, HH), jnp.float32),
        ],
    )(agg2, x, f2w, f2b, ow, ob, inw)


def _tc_node2_body(agg2_ref, x_ref, f2w_ref, f2b_ref, ow_ref, ob_ref,
                   w1_ref, b1_ref, w2_ref, b2_ref, out_ref):
    xn = _node_update(agg2_ref, x_ref, f2w_ref, f2b_ref, ow_ref, ob_ref)
    h = _ssp(jnp.dot(xn, w1_ref[...], preferred_element_type=jnp.float32) + b1_ref[...])
    out_ref[...] = jnp.dot(h, w2_ref[...], preferred_element_type=jnp.float32) + b2_ref[...]


def _tc_node2(agg2, x, f2w, f2b, ow, ob, w1, b1, w2, b2):
    return pl.pallas_call(
        _tc_node2_body,
        grid=(N // BN,),
        in_specs=[
            pl.BlockSpec((2, BN, HH), lambda n: (0, n, 0)),
            pl.BlockSpec((BN, H), lambda n: (n, 0)),
            pl.BlockSpec((H, H), lambda n: (0, 0)),
            pl.BlockSpec((H,), lambda n: (0,)),
            pl.BlockSpec((H, H), lambda n: (0, 0)),
            pl.BlockSpec((H,), lambda n: (0,)),
            pl.BlockSpec((H, H // 2), lambda n: (0, 0)),
            pl.BlockSpec((H // 2,), lambda n: (0,)),
            pl.BlockSpec((H // 2, 3), lambda n: (0, 0)),
            pl.BlockSpec((3,), lambda n: (0,)),
        ],
        out_specs=pl.BlockSpec((BN, 3), lambda n: (n, 0)),
        out_shape=jax.ShapeDtypeStruct((N, 3), jnp.float32),
    )(agg2, x, f2w, f2b, ow, ob, w1, b1, w2, b2)


def kernel(z, pos, edge_index, batch, params):
    i = edge_index[0].astype(jnp.int32)
    j = edge_index[1].astype(jnp.int32)
    pad = E_D - E
    i_d = jnp.concatenate([i, N + (jnp.arange(pad, dtype=jnp.int32) % (NP - N))])
    j_d = jnp.concatenate([j, jnp.zeros((pad,), jnp.int32)])

    qd = _sc_dist(pos[:, 0], pos[:, 1], pos[:, 2], i_d, j_d)
    qd2 = qd.reshape(E_D // GW, GW)
    ij2 = jnp.bitwise_or(i_d, jnp.left_shift(j_d, 16)).reshape(E_D // GW, GW)

    inter = params['interactions']
    fw1 = jnp.stack([p['fw1'] for p in inter])
    fb1 = jnp.stack([p['fb1'] for p in inter])
    fw2 = jnp.stack([p['fw2'] for p in inter])
    fb2 = jnp.stack([p['fb2'] for p in inter])
    tab = _tc_table(fw1, fb1, fw2, fb2)

    x, y2 = _tc_node0(z.astype(jnp.int32).reshape(N // BN, 1, BN),
                      params['emb'], inter[0]['inw'])
    score = None
    for t in range(NI):
        ytab = y2.reshape(NC * N, HH)
        agg2 = _sc_edge(t, ytab, tab, qd2, ij2)
        p = inter[t]
        if t < NI - 1:
            x, y2 = _tc_node(agg2, x, p['f2w'], p['f2b'], p['ow'], p['ob'],
                             inter[t + 1]['inw'])
        else:
            score = _tc_node2(agg2, x, p['f2w'], p['f2b'], p['ow'], p['ob'],
                              params['out_w1'], params['out_b1'],
                              params['out_w2'], params['out_b2'])
    return score
